# baseline (speedup 1.0000x reference)
import functools

import jax
import jax.numpy as jnp
from jax import lax
from jax.experimental import pallas as pl
from jax.experimental.pallas import tpu as pltpu
from jax.experimental.pallas import tpu_sc as plsc

_DIM = 32
_MBLK = 32768


def _row_mean_body(x_ref, o_ref):
    o_ref[...] = jnp.sum(x_ref[...], axis=0) * (1.0 / _DIM)


def _row_means(table_t):
    rows = table_t.shape[1]
    grid = (rows + _MBLK - 1) // _MBLK
    return pl.pallas_call(
        _row_mean_body,
        grid=(grid,),
        in_specs=[pl.BlockSpec((_DIM, _MBLK), lambda i: (0, i))],
        out_specs=pl.BlockSpec((_MBLK,), lambda i: (i,)),
        out_shape=jax.ShapeDtypeStruct((grid * _MBLK,), jnp.float32),
    )(table_t)


def _gather_means(means, idx_a, idx_p, idx_n):
    info = plsc.get_sparse_core_info()
    nw = info.num_cores * info.num_subcores
    na, nn = idx_a.shape[0], idx_n.shape[0]
    apw, npw = na // nw, nn // nw
    tpw = 2 * apw + npw
    mesh = plsc.VectorSubcoreMesh(core_axis_name="c", subcore_axis_name="s")

    @functools.partial(
        pl.kernel, mesh=mesh,
        out_type=[jax.ShapeDtypeStruct((na,), jnp.float32),
                  jax.ShapeDtypeStruct((na,), jnp.float32),
                  jax.ShapeDtypeStruct((nn,), jnp.float32)],
        scratch_types=[
            pltpu.VMEM((tpw,), jnp.int32),
            pltpu.VMEM((tpw,), jnp.float32),
            pltpu.SemaphoreType.DMA,
        ],
    )
    def gather_kernel(means_hbm, a_hbm, p_hbm, n_hbm,
                      oa_hbm, op_hbm, on_hbm, idx_v, vals_v, sem):
        wid = lax.axis_index("s") * info.num_cores + lax.axis_index("c")
        ab, nb = wid * apw, wid * npw
        pltpu.sync_copy(a_hbm.at[pl.ds(ab, apw)], idx_v.at[pl.ds(0, apw)])
        pltpu.sync_copy(p_hbm.at[pl.ds(ab, apw)],
                        idx_v.at[pl.ds(apw, apw)])
        pltpu.sync_copy(n_hbm.at[pl.ds(nb, npw)],
                        idx_v.at[pl.ds(2 * apw, npw)])
        pltpu.async_copy(means_hbm.at[idx_v], vals_v, sem).wait()
        pltpu.sync_copy(vals_v.at[pl.ds(0, apw)], oa_hbm.at[pl.ds(ab, apw)])
        pltpu.sync_copy(vals_v.at[pl.ds(apw, apw)],
                        op_hbm.at[pl.ds(ab, apw)])
        pltpu.sync_copy(vals_v.at[pl.ds(2 * apw, npw)],
                        on_hbm.at[pl.ds(nb, npw)])

    return gather_kernel(means, idx_a, idx_p, idx_n)


def _norm_body(a_ref, p_ref, n_ref, oa_ref, op_ref, on_ref):
    for x_ref, o_ref in ((a_ref, oa_ref), (p_ref, op_ref), (n_ref, on_ref)):
        x = x_ref[...]
        ss = jnp.sum(x * x, axis=0, keepdims=True)
        o_ref[...] = x / jnp.sqrt(ss)


def _normalize(va, vp, vn):
    seq = va.shape[0]
    ca, cn = va.shape[2], vn.shape[2]
    grid = 8
    ba, bn = ca // grid, cn // grid
    spec_a = pl.BlockSpec((seq, 1, ba), lambda i: (0, 0, i))
    spec_n = pl.BlockSpec((seq, 1, bn), lambda i: (0, 0, i))
    return pl.pallas_call(
        _norm_body,
        grid=(grid,),
        in_specs=[spec_a, spec_a, spec_n],
        out_specs=[spec_a, spec_a, spec_n],
        out_shape=[jax.ShapeDtypeStruct((seq, 1, ca), jnp.float32),
                   jax.ShapeDtypeStruct((seq, 1, ca), jnp.float32),
                   jax.ShapeDtypeStruct((seq, 1, cn), jnp.float32)],
    )(va, vp, vn)


def kernel(anchor_input_ids, positive_input_ids, negative_input_ids,
           embedding_table):
    batch, seq = anchor_input_ids.shape
    num_neg = negative_input_ids.shape[1]
    na = batch * seq

    means = _row_means(embedding_table.T)
    nt = negative_input_ids.transpose(2, 1, 0)
    nt = nt.reshape(seq, num_neg, batch // 128, 128).transpose(0, 2, 1, 3)
    fa, fp, fn = _gather_means(means,
                               anchor_input_ids.T.reshape(-1),
                               positive_input_ids.T.reshape(-1),
                               nt.reshape(-1))

    va = fa.reshape(seq, 1, batch)
    vp = fp.reshape(seq, 1, batch)
    vn = fn.reshape(seq, 1, num_neg * batch)
    oa, op_, on = _normalize(va, vp, vn)

    anchor = oa.transpose(2, 0, 1)
    positive = op_.transpose(2, 0, 1)
    negative = (on.reshape(seq, batch // 128, num_neg, 128)
                .transpose(1, 3, 2, 0).reshape(batch, num_neg, seq))
    return (anchor, positive, negative)


# speedup vs baseline: 1.0182x; 1.0182x over previous
import functools

import jax
import jax.numpy as jnp
from jax import lax
from jax.experimental import pallas as pl
from jax.experimental.pallas import tpu as pltpu
from jax.experimental.pallas import tpu_sc as plsc

_DIM = 32
_MBLK = 131072


def _row_mean_body(x_ref, o_ref):
    o_ref[...] = jnp.sum(x_ref[...], axis=0) * (1.0 / _DIM)


def _row_means(table_t):
    rows = table_t.shape[1]
    grid = (rows + _MBLK - 1) // _MBLK
    return pl.pallas_call(
        _row_mean_body,
        grid=(grid,),
        in_specs=[pl.BlockSpec((_DIM, _MBLK), lambda i: (0, i))],
        out_specs=pl.BlockSpec((_MBLK,), lambda i: (i,)),
        out_shape=jax.ShapeDtypeStruct((grid * _MBLK,), jnp.float32),
    )(table_t)


def _gather_means(means, idx_a, idx_p, idx_n):
    info = plsc.get_sparse_core_info()
    nw = info.num_cores * info.num_subcores
    na, nn = idx_a.shape[0], idx_n.shape[0]
    apw, npw = na // nw, nn // nw
    tpw = 2 * apw + npw
    mesh = plsc.VectorSubcoreMesh(core_axis_name="c", subcore_axis_name="s")

    @functools.partial(
        pl.kernel, mesh=mesh,
        out_type=[jax.ShapeDtypeStruct((na,), jnp.float32),
                  jax.ShapeDtypeStruct((na,), jnp.float32),
                  jax.ShapeDtypeStruct((nn,), jnp.float32)],
        scratch_types=[
            pltpu.VMEM((tpw,), jnp.int32),
            pltpu.VMEM((tpw,), jnp.float32),
            pltpu.SemaphoreType.DMA,
        ],
    )
    def gather_kernel(means_hbm, a_hbm, p_hbm, n_hbm,
                      oa_hbm, op_hbm, on_hbm, idx_v, vals_v, sem):
        wid = lax.axis_index("s") * info.num_cores + lax.axis_index("c")
        ab, nb = wid * apw, wid * npw
        pltpu.sync_copy(a_hbm.at[pl.ds(ab, apw)], idx_v.at[pl.ds(0, apw)])
        pltpu.sync_copy(p_hbm.at[pl.ds(ab, apw)],
                        idx_v.at[pl.ds(apw, apw)])
        pltpu.sync_copy(n_hbm.at[pl.ds(nb, npw)],
                        idx_v.at[pl.ds(2 * apw, npw)])
        pltpu.async_copy(means_hbm.at[idx_v], vals_v, sem).wait()
        pltpu.sync_copy(vals_v.at[pl.ds(0, apw)], oa_hbm.at[pl.ds(ab, apw)])
        pltpu.sync_copy(vals_v.at[pl.ds(apw, apw)],
                        op_hbm.at[pl.ds(ab, apw)])
        pltpu.sync_copy(vals_v.at[pl.ds(2 * apw, npw)],
                        on_hbm.at[pl.ds(nb, npw)])

    return gather_kernel(means, idx_a, idx_p, idx_n)


def _norm_body(a_ref, p_ref, n_ref, oa_ref, op_ref, on_ref):
    for x_ref, o_ref in ((a_ref, oa_ref), (p_ref, op_ref), (n_ref, on_ref)):
        x = x_ref[...]
        ss = jnp.sum(x * x, axis=0, keepdims=True)
        o_ref[...] = x / jnp.sqrt(ss)


def _normalize(va, vp, vn):
    seq = va.shape[0]
    ca, cn = va.shape[2], vn.shape[2]
    grid = 8
    ba, bn = ca // grid, cn // grid
    spec_a = pl.BlockSpec((seq, 1, ba), lambda i: (0, 0, i))
    spec_n = pl.BlockSpec((seq, 1, bn), lambda i: (0, 0, i))
    return pl.pallas_call(
        _norm_body,
        grid=(grid,),
        in_specs=[spec_a, spec_a, spec_n],
        out_specs=[spec_a, spec_a, spec_n],
        out_shape=[jax.ShapeDtypeStruct((seq, 1, ca), jnp.float32),
                   jax.ShapeDtypeStruct((seq, 1, ca), jnp.float32),
                   jax.ShapeDtypeStruct((seq, 1, cn), jnp.float32)],
    )(va, vp, vn)


def kernel(anchor_input_ids, positive_input_ids, negative_input_ids,
           embedding_table):
    batch, seq = anchor_input_ids.shape
    num_neg = negative_input_ids.shape[1]
    na = batch * seq

    means = _row_means(embedding_table.T)
    nt = negative_input_ids.transpose(2, 1, 0)
    nt = nt.reshape(seq, num_neg, batch // 128, 128).transpose(0, 2, 1, 3)
    fa, fp, fn = _gather_means(means,
                               anchor_input_ids.T.reshape(-1),
                               positive_input_ids.T.reshape(-1),
                               nt.reshape(-1))

    va = fa.reshape(seq, 1, batch)
    vp = fp.reshape(seq, 1, batch)
    vn = fn.reshape(seq, 1, num_neg * batch)
    oa, op_, on = _normalize(va, vp, vn)

    anchor = oa.transpose(2, 0, 1)
    positive = op_.transpose(2, 0, 1)
    negative = (on.reshape(seq, batch // 128, num_neg, 128)
                .transpose(1, 3, 2, 0).reshape(batch, num_neg, seq))
    return (anchor, positive, negative)


# speedup vs baseline: 1.0187x; 1.0005x over previous
import functools

import jax
import jax.numpy as jnp
from jax import lax
from jax.experimental import pallas as pl
from jax.experimental.pallas import tpu as pltpu
from jax.experimental.pallas import tpu_sc as plsc

_DIM = 32
_MBLK = 65536


def _row_mean_body(x_ref, o_ref):
    o_ref[...] = jnp.sum(x_ref[...], axis=0) * (1.0 / _DIM)


def _row_means(table_t):
    rows = table_t.shape[1]
    grid = (rows + _MBLK - 1) // _MBLK
    return pl.pallas_call(
        _row_mean_body,
        grid=(grid,),
        in_specs=[pl.BlockSpec((_DIM, _MBLK), lambda i: (0, i))],
        out_specs=pl.BlockSpec((_MBLK,), lambda i: (i,)),
        out_shape=jax.ShapeDtypeStruct((grid * _MBLK,), jnp.float32),
    )(table_t)


def _gather_means(means, idx_a, idx_p, idx_n):
    info = plsc.get_sparse_core_info()
    nw = info.num_cores * info.num_subcores
    na, nn = idx_a.shape[0], idx_n.shape[0]
    apw, npw = na // nw, nn // nw
    tpw = 2 * apw + npw
    mesh = plsc.VectorSubcoreMesh(core_axis_name="c", subcore_axis_name="s")

    @functools.partial(
        pl.kernel, mesh=mesh,
        out_type=[jax.ShapeDtypeStruct((na,), jnp.float32),
                  jax.ShapeDtypeStruct((na,), jnp.float32),
                  jax.ShapeDtypeStruct((nn,), jnp.float32)],
        scratch_types=[
            pltpu.VMEM((tpw,), jnp.int32),
            pltpu.VMEM((tpw,), jnp.float32),
            pltpu.SemaphoreType.DMA,
            pltpu.SemaphoreType.DMA,
        ],
    )
    def gather_kernel(means_hbm, a_hbm, p_hbm, n_hbm,
                      oa_hbm, op_hbm, on_hbm, idx_v, vals_v, sem_a, sem_b):
        wid = lax.axis_index("s") * info.num_cores + lax.axis_index("c")
        ab, nb = wid * apw, wid * npw
        half = tpw // 2
        nh = half - 2 * apw
        pltpu.sync_copy(a_hbm.at[pl.ds(ab, apw)], idx_v.at[pl.ds(0, apw)])
        pltpu.sync_copy(p_hbm.at[pl.ds(ab, apw)],
                        idx_v.at[pl.ds(apw, apw)])
        pltpu.sync_copy(n_hbm.at[pl.ds(nb, nh)],
                        idx_v.at[pl.ds(2 * apw, nh)])
        da = pltpu.async_copy(means_hbm.at[idx_v.at[pl.ds(0, half)]],
                              vals_v.at[pl.ds(0, half)], sem_a)
        pltpu.sync_copy(n_hbm.at[pl.ds(nb + nh, npw - nh)],
                        idx_v.at[pl.ds(half, npw - nh)])
        db = pltpu.async_copy(means_hbm.at[idx_v.at[pl.ds(half, half)]],
                              vals_v.at[pl.ds(half, half)], sem_b)
        da.wait()
        pltpu.sync_copy(vals_v.at[pl.ds(0, apw)], oa_hbm.at[pl.ds(ab, apw)])
        pltpu.sync_copy(vals_v.at[pl.ds(apw, apw)],
                        op_hbm.at[pl.ds(ab, apw)])
        pltpu.sync_copy(vals_v.at[pl.ds(2 * apw, nh)],
                        on_hbm.at[pl.ds(nb, nh)])
        db.wait()
        pltpu.sync_copy(vals_v.at[pl.ds(half, npw - nh)],
                        on_hbm.at[pl.ds(nb + nh, npw - nh)])

    return gather_kernel(means, idx_a, idx_p, idx_n)


def _norm_body(a_ref, p_ref, n_ref, oa_ref, op_ref, on_ref):
    for x_ref, o_ref in ((a_ref, oa_ref), (p_ref, op_ref), (n_ref, on_ref)):
        x = x_ref[...]
        ss = jnp.sum(x * x, axis=0, keepdims=True)
        o_ref[...] = x / jnp.sqrt(ss)


def _normalize(va, vp, vn):
    seq = va.shape[0]
    ca, cn = va.shape[2], vn.shape[2]
    grid = 8
    ba, bn = ca // grid, cn // grid
    spec_a = pl.BlockSpec((seq, 1, ba), lambda i: (0, 0, i))
    spec_n = pl.BlockSpec((seq, 1, bn), lambda i: (0, 0, i))
    return pl.pallas_call(
        _norm_body,
        grid=(grid,),
        in_specs=[spec_a, spec_a, spec_n],
        out_specs=[spec_a, spec_a, spec_n],
        out_shape=[jax.ShapeDtypeStruct((seq, 1, ca), jnp.float32),
                   jax.ShapeDtypeStruct((seq, 1, ca), jnp.float32),
                   jax.ShapeDtypeStruct((seq, 1, cn), jnp.float32)],
    )(va, vp, vn)


def kernel(anchor_input_ids, positive_input_ids, negative_input_ids,
           embedding_table):
    batch, seq = anchor_input_ids.shape
    num_neg = negative_input_ids.shape[1]
    na = batch * seq

    means = _row_means(embedding_table.T)
    nt = negative_input_ids.transpose(2, 1, 0)
    nt = nt.reshape(seq, num_neg, batch // 128, 128).transpose(0, 2, 1, 3)
    fa, fp, fn = _gather_means(means,
                               anchor_input_ids.T.reshape(-1),
                               positive_input_ids.T.reshape(-1),
                               nt.reshape(-1))

    va = fa.reshape(seq, 1, batch)
    vp = fp.reshape(seq, 1, batch)
    vn = fn.reshape(seq, 1, num_neg * batch)
    oa, op_, on = _normalize(va, vp, vn)

    anchor = oa.transpose(2, 0, 1)
    positive = op_.transpose(2, 0, 1)
    negative = (on.reshape(seq, batch // 128, num_neg, 128)
                .transpose(1, 3, 2, 0).reshape(batch, num_neg, seq))
    return (anchor, positive, negative)


# speedup vs baseline: 1.0336x; 1.0147x over previous
import functools

import jax
import jax.numpy as jnp
from jax import lax
from jax.experimental import pallas as pl
from jax.experimental.pallas import tpu as pltpu
from jax.experimental.pallas import tpu_sc as plsc

_DIM = 32
_MBLK = 65536


def _row_mean_body(x_ref, o_ref):
    o_ref[...] = jnp.sum(x_ref[...], axis=0) * (1.0 / _DIM)


def _row_means(table_t):
    rows = table_t.shape[1]
    grid = (rows + _MBLK - 1) // _MBLK
    return pl.pallas_call(
        _row_mean_body,
        grid=(grid,),
        in_specs=[pl.BlockSpec((_DIM, _MBLK), lambda i: (0, i))],
        out_specs=pl.BlockSpec((_MBLK,), lambda i: (i,)),
        out_shape=jax.ShapeDtypeStruct((grid * _MBLK,), jnp.float32),
    )(table_t)


def _gather_means(means, idx_a, idx_p, idx_n):
    info = plsc.get_sparse_core_info()
    nw = info.num_cores * info.num_subcores
    na, nn = idx_a.shape[0], idx_n.shape[0]
    apw, npw = na // nw, nn // nw
    tpw = 2 * apw + npw
    mesh = plsc.VectorSubcoreMesh(core_axis_name="c", subcore_axis_name="s")

    @functools.partial(
        pl.kernel, mesh=mesh,
        out_type=[jax.ShapeDtypeStruct((na,), jnp.float32),
                  jax.ShapeDtypeStruct((na,), jnp.float32),
                  jax.ShapeDtypeStruct((nn,), jnp.float32)],
        scratch_types=[
            pltpu.VMEM((tpw,), jnp.int32),
            pltpu.VMEM((tpw,), jnp.float32),
            pltpu.SemaphoreType.DMA,
        ],
    )
    def gather_kernel(means_hbm, a_hbm, p_hbm, n_hbm,
                      oa_hbm, op_hbm, on_hbm, idx_v, vals_v, sem):
        wid = lax.axis_index("s") * info.num_cores + lax.axis_index("c")
        ab, nb = wid * apw, wid * npw
        pltpu.sync_copy(a_hbm.at[pl.ds(ab, apw)], idx_v.at[pl.ds(0, apw)])
        pltpu.sync_copy(p_hbm.at[pl.ds(ab, apw)],
                        idx_v.at[pl.ds(apw, apw)])
        pltpu.sync_copy(n_hbm.at[pl.ds(nb, npw)],
                        idx_v.at[pl.ds(2 * apw, npw)])
        pltpu.async_copy(means_hbm.at[idx_v], vals_v, sem).wait()
        pltpu.sync_copy(vals_v.at[pl.ds(0, apw)], oa_hbm.at[pl.ds(ab, apw)])
        pltpu.sync_copy(vals_v.at[pl.ds(apw, apw)],
                        op_hbm.at[pl.ds(ab, apw)])
        pltpu.sync_copy(vals_v.at[pl.ds(2 * apw, npw)],
                        on_hbm.at[pl.ds(nb, npw)])

    return gather_kernel(means, idx_a, idx_p, idx_n)


def _norm_body(a_ref, p_ref, n_ref, oa_ref, op_ref, on_ref):
    for x_ref, o_ref in ((a_ref, oa_ref), (p_ref, op_ref), (n_ref, on_ref)):
        x = x_ref[...]
        ss = jnp.sum(x * x, axis=0, keepdims=True)
        o_ref[...] = x / jnp.sqrt(ss)


def _normalize(va, vp, vn):
    seq = va.shape[0]
    ca, cn = va.shape[2], vn.shape[2]
    grid = 8
    ba, bn = ca // grid, cn // grid
    spec_a = pl.BlockSpec((seq, 1, ba), lambda i: (0, 0, i))
    spec_n = pl.BlockSpec((seq, 1, bn), lambda i: (0, 0, i))
    return pl.pallas_call(
        _norm_body,
        grid=(grid,),
        in_specs=[spec_a, spec_a, spec_n],
        out_specs=[spec_a, spec_a, spec_n],
        out_shape=[jax.ShapeDtypeStruct((seq, 1, ca), jnp.float32),
                   jax.ShapeDtypeStruct((seq, 1, ca), jnp.float32),
                   jax.ShapeDtypeStruct((seq, 1, cn), jnp.float32)],
    )(va, vp, vn)


def kernel(anchor_input_ids, positive_input_ids, negative_input_ids,
           embedding_table):
    batch, seq = anchor_input_ids.shape
    num_neg = negative_input_ids.shape[1]
    na = batch * seq

    means = _row_means(embedding_table.T)
    nt = negative_input_ids.transpose(2, 1, 0)
    nt = nt.reshape(seq, num_neg, batch // 128, 128).transpose(0, 2, 1, 3)
    fa, fp, fn = _gather_means(means,
                               anchor_input_ids.T.reshape(-1),
                               positive_input_ids.T.reshape(-1),
                               nt.reshape(-1))

    va = fa.reshape(seq, 1, batch)
    vp = fp.reshape(seq, 1, batch)
    vn = fn.reshape(seq, 1, num_neg * batch)
    oa, op_, on = _normalize(va, vp, vn)

    anchor = oa.transpose(2, 0, 1)
    positive = op_.transpose(2, 0, 1)
    negative = (on.reshape(seq, batch // 128, num_neg, 128)
                .transpose(1, 3, 2, 0).reshape(batch, num_neg, seq))
    return (anchor, positive, negative)
